# baseline (speedup 1.0000x reference)
import jax
import jax.numpy as jnp
from jax.experimental import pallas as pl

BATCH = 4
SEQ_LEN = 2048
EMBED_DIM = 768
SEQ_BLOCK = 2048


def _add_kernel(x_ref, pos_ref, o_ref):
    o_ref[...] = x_ref[...] + pos_ref[...]


def kernel(x, pos_table):
    grid = (2,)
    return pl.pallas_call(
        _add_kernel,
        grid=grid,
        in_specs=[
            pl.BlockSpec((BATCH, SEQ_LEN, EMBED_DIM // 2), lambda e: (0, 0, e)),
            pl.BlockSpec((SEQ_LEN, EMBED_DIM // 2), lambda e: (0, e)),
        ],
        out_specs=pl.BlockSpec((BATCH, SEQ_LEN, EMBED_DIM // 2), lambda e: (0, 0, e)),
        out_shape=jax.ShapeDtypeStruct(x.shape, x.dtype),
    )(x, pos_table)


# speedup vs baseline: 1.0704x; 1.0704x over previous
import jax
import jax.numpy as jnp
from jax.experimental import pallas as pl

BATCH = 4
SEQ_LEN = 2048
EMBED_DIM = 768
SEQ_BLOCK = 2048


def _add_kernel(x_ref, pos_ref, o_ref):
    o_ref[...] = x_ref[...] + pos_ref[...]


def kernel(x, pos_table):
    grid = (BATCH // 2,)
    return pl.pallas_call(
        _add_kernel,
        grid=grid,
        in_specs=[
            pl.BlockSpec((2, SEQ_LEN, EMBED_DIM), lambda b: (b, 0, 0)),
            pl.BlockSpec((SEQ_LEN, EMBED_DIM), lambda b: (0, 0)),
        ],
        out_specs=pl.BlockSpec((2, SEQ_LEN, EMBED_DIM), lambda b: (b, 0, 0)),
        out_shape=jax.ShapeDtypeStruct(x.shape, x.dtype),
    )(x, pos_table)
